# baseline (speedup 1.0000x reference)
import math

import jax
import jax.numpy as jnp
from jax.experimental import pallas as pl

S, D = 2048, 768
H, DH = 12, 64
E, DFF = 16, 3072
C = 2 * math.ceil(S / E)
LN_EPS = 1e-5
NEG = -1e30
SCALE = 0.8

_INTERPRET = False

QT = 1024
TT = 256


def _ln(x, g, b):
    mu = jnp.mean(x, axis=-1, keepdims=True)
    xc = x - mu
    var = jnp.mean(xc * xc, axis=-1, keepdims=True)
    return xc * jax.lax.rsqrt(var + LN_EPS) * g + b


def _dot_t(x, w):
    return jax.lax.dot_general(x, w, (((1,), (1,)), ((), ())),
                               preferred_element_type=jnp.float32)


def _k1(h_ref, g_ref, b_ref, qw_ref, qb_ref, kw_ref, kb_ref, vw_ref, vb_ref,
        q_ref, k_ref, v_ref):
    x = _ln(h_ref[...], g_ref[...], b_ref[...])
    q_ref[...] = (_dot_t(x, qw_ref[...]) + qb_ref[...]) * (1.0 / math.sqrt(DH))
    k_ref[...] = _dot_t(x, kw_ref[...]) + kb_ref[...]
    v_ref[...] = _dot_t(x, vw_ref[...]) + vb_ref[...]


def _k2(q_ref, k_ref, v_ref, o_ref):
    qi = pl.program_id(1)
    s = _dot_t(q_ref[0], k_ref[0])
    ri = jax.lax.broadcasted_iota(jnp.int32, (QT, S), 0) + qi * QT
    ci = jax.lax.broadcasted_iota(jnp.int32, (QT, S), 1)
    s = jnp.where(ri >= ci, s, NEG)
    m = jnp.max(s, axis=1, keepdims=True)
    p = jnp.exp(s - m)
    p = p / jnp.sum(p, axis=1, keepdims=True)
    o_ref[0] = jnp.dot(p, v_ref[0], preferred_element_type=jnp.float32)


def _k3(ao_ref, ow_ref, ob_ref, res_ref, g_ref, b_ref, rw_ref,
        h2_ref, x2_ref, lg_ref):
    h2 = _dot_t(ao_ref[...], ow_ref[...]) + ob_ref[...] + res_ref[...]
    h2_ref[...] = h2
    x2 = _ln(h2, g_ref[...], b_ref[...])
    x2_ref[...] = x2
    lg_ref[...] = _dot_t(x2, rw_ref[...])


def _k4(lg_ref, route_ref):
    lg = lg_ref[...]
    eidx = jax.lax.broadcasted_iota(jnp.int32, (S, E), 1)
    mx1 = jnp.max(lg, axis=1, keepdims=True)
    t1 = jnp.min(jnp.where(lg == mx1, eidx, E), axis=1, keepdims=True)
    m1 = (eidx == t1).astype(jnp.float32)
    lg2 = jnp.where(m1 > 0, NEG, lg)
    mx2 = jnp.max(lg2, axis=1, keepdims=True)
    t2 = jnp.min(jnp.where(lg2 == mx2, eidx, E), axis=1, keepdims=True)
    m2 = (eidx == t2).astype(jnp.float32)
    ex = jnp.exp(lg - mx1)
    probs = ex / jnp.sum(ex, axis=1, keepdims=True)
    ri = jax.lax.broadcasted_iota(jnp.int32, (S, S), 0)
    ci = jax.lax.broadcasted_iota(jnp.int32, (S, S), 1)
    lt = (ri >= ci).astype(jnp.float32)
    cs1 = jax.lax.dot(lt, m1, precision=jax.lax.Precision.HIGHEST,
                      preferred_element_type=jnp.float32)
    cs2 = jax.lax.dot(lt, m2, precision=jax.lax.Precision.HIGHEST,
                      preferred_element_type=jnp.float32)
    loc1 = cs1 - 1.0
    loc2 = cs2 - 1.0 + jnp.sum(m1, axis=0, keepdims=True)
    k1m = m1 * (loc1 < C).astype(jnp.float32)
    k2m = m2 * (loc2 < C).astype(jnp.float32)
    p1 = jnp.sum(probs * k1m, axis=1, keepdims=True)
    p2 = jnp.sum(probs * k2m, axis=1, keepdims=True)
    denom = jnp.maximum(p1 + p2, jnp.finfo(jnp.float32).eps)
    w1 = p1 / denom * SCALE
    w2 = p2 / denom * SCALE
    kept1 = jnp.sum(k1m, axis=1, keepdims=True)
    kept2 = jnp.sum(k2m, axis=1, keepdims=True)
    ef = eidx.astype(jnp.float32)
    flat1 = jnp.where(kept1 > 0,
                      jnp.sum(m1 * (ef * C + loc1), axis=1, keepdims=True),
                      -1.0)
    flat2 = jnp.where(kept2 > 0,
                      jnp.sum(m2 * (ef * C + loc2), axis=1, keepdims=True),
                      -1.0)
    pad = jnp.zeros((S, 124), jnp.float32)
    route_ref[...] = jnp.concatenate([flat1, flat2, w1, w2, pad], axis=1)


def _k5(x2_ref, route_ref, buf_ref):
    e = pl.program_id(0)
    f1 = route_ref[:, 0:1].astype(jnp.int32)
    f2 = route_ref[:, 1:2].astype(jnp.int32)
    sl = jax.lax.broadcasted_iota(jnp.int32, (S, C), 1) + e * C
    p = jnp.logical_or(f1 == sl, f2 == sl).astype(jnp.bfloat16)
    buf_ref[...] = jax.lax.dot_general(
        p, x2_ref[...].astype(jnp.bfloat16), (((0,), (0,)), ((), ())),
        preferred_element_type=jnp.float32)


def _k6(buf_ref, w1_ref, b1_ref, w2_ref, b2_ref, out_ref):
    h = jnp.dot(buf_ref[...].astype(jnp.bfloat16),
                w1_ref[0].astype(jnp.bfloat16),
                preferred_element_type=jnp.float32) + b1_ref[0]
    h = h * 0.5 * (1.0 + jax.lax.erf(h * (1.0 / math.sqrt(2.0))))
    out_ref[...] = jnp.dot(h.astype(jnp.bfloat16),
                           w2_ref[0].astype(jnp.bfloat16),
                           preferred_element_type=jnp.float32) + b2_ref[0]


def _k7(route_ref, h2f_ref, res_ref, out_ref):
    f1 = route_ref[:, 0:1].astype(jnp.int32)
    f2 = route_ref[:, 1:2].astype(jnp.int32)
    w1 = route_ref[:, 2:3]
    w2 = route_ref[:, 3:4]
    sl = jax.lax.broadcasted_iota(jnp.int32, (TT, E * C), 1)
    ce = (w1 * (f1 == sl).astype(jnp.float32)
          + w2 * (f2 == sl).astype(jnp.float32))
    out_ref[...] = res_ref[...] + jnp.dot(
        ce.astype(jnp.bfloat16), h2f_ref[...].astype(jnp.bfloat16),
        preferred_element_type=jnp.float32)


def kernel(hidden_states, ln1_g, ln1_b, q_w, q_b, k_w, k_b, v_w, v_b,
           o_w, o_b, ln2_g, ln2_b, router_w, w1, b1, w2, b2):
    f32 = jnp.float32
    h = hidden_states.reshape(S, D)
    r2 = lambda a: a.reshape(1, D)

    nt = S // TT
    q, k, v = pl.pallas_call(
        _k1,
        grid=(nt,),
        in_specs=[
            pl.BlockSpec((TT, D), lambda i: (i, 0)),
            pl.BlockSpec((1, D), lambda i: (0, 0)),
            pl.BlockSpec((1, D), lambda i: (0, 0)),
            pl.BlockSpec((D, D), lambda i: (0, 0)),
            pl.BlockSpec((1, D), lambda i: (0, 0)),
            pl.BlockSpec((D, D), lambda i: (0, 0)),
            pl.BlockSpec((1, D), lambda i: (0, 0)),
            pl.BlockSpec((D, D), lambda i: (0, 0)),
            pl.BlockSpec((1, D), lambda i: (0, 0)),
        ],
        out_specs=[pl.BlockSpec((TT, D), lambda i: (i, 0))] * 3,
        out_shape=[jax.ShapeDtypeStruct((S, D), f32)] * 3,
        interpret=_INTERPRET,
    )(h, r2(ln1_g), r2(ln1_b), q_w, r2(q_b), k_w, r2(k_b), v_w, r2(v_b))

    split = lambda t: t.reshape(S, H, DH).transpose(1, 0, 2)
    ao = pl.pallas_call(
        _k2,
        grid=(H, S // QT),
        in_specs=[
            pl.BlockSpec((1, QT, DH), lambda hh, qi: (hh, qi, 0)),
            pl.BlockSpec((1, S, DH), lambda hh, qi: (hh, 0, 0)),
            pl.BlockSpec((1, S, DH), lambda hh, qi: (hh, 0, 0)),
        ],
        out_specs=pl.BlockSpec((1, QT, DH), lambda hh, qi: (hh, qi, 0)),
        out_shape=jax.ShapeDtypeStruct((H, S, DH), f32),
        interpret=_INTERPRET,
    )(split(q), split(k), split(v))
    ao = ao.transpose(1, 0, 2).reshape(S, D)

    h2, x2, lg = pl.pallas_call(
        _k3,
        grid=(nt,),
        in_specs=[
            pl.BlockSpec((TT, D), lambda i: (i, 0)),
            pl.BlockSpec((D, D), lambda i: (0, 0)),
            pl.BlockSpec((1, D), lambda i: (0, 0)),
            pl.BlockSpec((TT, D), lambda i: (i, 0)),
            pl.BlockSpec((1, D), lambda i: (0, 0)),
            pl.BlockSpec((1, D), lambda i: (0, 0)),
            pl.BlockSpec((E, D), lambda i: (0, 0)),
        ],
        out_specs=[
            pl.BlockSpec((TT, D), lambda i: (i, 0)),
            pl.BlockSpec((TT, D), lambda i: (i, 0)),
            pl.BlockSpec((TT, E), lambda i: (i, 0)),
        ],
        out_shape=[
            jax.ShapeDtypeStruct((S, D), f32),
            jax.ShapeDtypeStruct((S, D), f32),
            jax.ShapeDtypeStruct((S, E), f32),
        ],
        interpret=_INTERPRET,
    )(ao, o_w, r2(o_b), h, r2(ln2_g), r2(ln2_b), router_w)

    route = pl.pallas_call(
        _k4,
        in_specs=[pl.BlockSpec((S, E), lambda: (0, 0))],
        out_specs=pl.BlockSpec((S, 128), lambda: (0, 0)),
        out_shape=jax.ShapeDtypeStruct((S, 128), f32),
        interpret=_INTERPRET,
    )(lg)

    buf = pl.pallas_call(
        _k5,
        grid=(E,),
        in_specs=[
            pl.BlockSpec((S, D), lambda e: (0, 0)),
            pl.BlockSpec((S, 128), lambda e: (0, 0)),
        ],
        out_specs=pl.BlockSpec((C, D), lambda e: (e, 0)),
        out_shape=jax.ShapeDtypeStruct((E * C, D), f32),
        interpret=_INTERPRET,
    )(x2, route)

    h2f = pl.pallas_call(
        _k6,
        grid=(E,),
        in_specs=[
            pl.BlockSpec((C, D), lambda e: (e, 0)),
            pl.BlockSpec((1, D, DFF), lambda e: (e, 0, 0)),
            pl.BlockSpec((1, 1, DFF), lambda e: (e, 0, 0)),
            pl.BlockSpec((1, DFF, D), lambda e: (e, 0, 0)),
            pl.BlockSpec((1, 1, D), lambda e: (e, 0, 0)),
        ],
        out_specs=pl.BlockSpec((C, D), lambda e: (e, 0)),
        out_shape=jax.ShapeDtypeStruct((E * C, D), f32),
        interpret=_INTERPRET,
    )(buf, w1, b1.reshape(E, 1, DFF), w2, b2.reshape(E, 1, D))

    out = pl.pallas_call(
        _k7,
        grid=(nt,),
        in_specs=[
            pl.BlockSpec((TT, 128), lambda i: (i, 0)),
            pl.BlockSpec((E * C, D), lambda i: (0, 0)),
            pl.BlockSpec((TT, D), lambda i: (i, 0)),
        ],
        out_specs=pl.BlockSpec((TT, D), lambda i: (i, 0)),
        out_shape=jax.ShapeDtypeStruct((S, D), f32),
        interpret=_INTERPRET,
    )(route, h2f, h2)

    return out.reshape(1, S, D)


# speedup vs baseline: 1.1703x; 1.1703x over previous
import math

import jax
import jax.numpy as jnp
from jax.experimental import pallas as pl

S, D = 2048, 768
H, DH = 12, 64
E, DFF = 16, 3072
C = 2 * math.ceil(S / E)
LN_EPS = 1e-5
NEG = -1e30
SCALE = 0.8

_INTERPRET = False

QT = 1024
TT = 256


def _ln(x, g, b):
    mu = jnp.mean(x, axis=-1, keepdims=True)
    xc = x - mu
    var = jnp.mean(xc * xc, axis=-1, keepdims=True)
    return xc * jax.lax.rsqrt(var + LN_EPS) * g + b


def _dot_t(x, w):
    return jax.lax.dot_general(x, w, (((1,), (1,)), ((), ())),
                               preferred_element_type=jnp.float32)


def _k1(h_ref, g_ref, b_ref, qw_ref, qb_ref, kw_ref, kb_ref, vw_ref, vb_ref,
        q_ref, k_ref, v_ref):
    x = _ln(h_ref[...], g_ref[...], b_ref[...])
    q = (_dot_t(x, qw_ref[...]) + qb_ref[...]) * (1.0 / math.sqrt(DH))
    k = _dot_t(x, kw_ref[...]) + kb_ref[...]
    v = _dot_t(x, vw_ref[...]) + vb_ref[...]
    for h in range(H):
        q_ref[h] = q[:, DH * h:DH * (h + 1)]
        k_ref[h] = k[:, DH * h:DH * (h + 1)]
        v_ref[h] = v[:, DH * h:DH * (h + 1)]


def _k2(q_ref, k_ref, v_ref, o_ref):
    qi = pl.program_id(1)
    s = _dot_t(q_ref[0], k_ref[0])
    ri = jax.lax.broadcasted_iota(jnp.int32, (QT, S), 0) + qi * QT
    ci = jax.lax.broadcasted_iota(jnp.int32, (QT, S), 1)
    s = jnp.where(ri >= ci, s, NEG)
    m = jnp.max(s, axis=1, keepdims=True)
    p = jnp.exp(s - m)
    p = p / jnp.sum(p, axis=1, keepdims=True)
    o_ref[0] = jnp.dot(p, v_ref[0], preferred_element_type=jnp.float32)


def _k3(ao_ref, ow_ref, ob_ref, res_ref, g_ref, b_ref, rw_ref,
        h2_ref, x2_ref, lg_ref):
    ao = jnp.concatenate([ao_ref[h] for h in range(H)], axis=1)
    h2 = _dot_t(ao, ow_ref[...]) + ob_ref[...] + res_ref[...]
    h2_ref[...] = h2
    x2 = _ln(h2, g_ref[...], b_ref[...])
    x2_ref[...] = x2
    lg_ref[...] = _dot_t(x2, rw_ref[...])


def _k4(lg_ref, route_ref):
    lg = lg_ref[...]
    eidx = jax.lax.broadcasted_iota(jnp.int32, (S, E), 1)
    mx1 = jnp.max(lg, axis=1, keepdims=True)
    t1 = jnp.min(jnp.where(lg == mx1, eidx, E), axis=1, keepdims=True)
    m1 = (eidx == t1).astype(jnp.float32)
    lg2 = jnp.where(m1 > 0, NEG, lg)
    mx2 = jnp.max(lg2, axis=1, keepdims=True)
    t2 = jnp.min(jnp.where(lg2 == mx2, eidx, E), axis=1, keepdims=True)
    m2 = (eidx == t2).astype(jnp.float32)
    ex = jnp.exp(lg - mx1)
    probs = ex / jnp.sum(ex, axis=1, keepdims=True)
    ri = jax.lax.broadcasted_iota(jnp.int32, (S, S), 0)
    ci = jax.lax.broadcasted_iota(jnp.int32, (S, S), 1)
    lt = (ri >= ci).astype(jnp.float32)
    cs1 = jax.lax.dot(lt, m1, precision=jax.lax.Precision.HIGHEST,
                      preferred_element_type=jnp.float32)
    cs2 = jax.lax.dot(lt, m2, precision=jax.lax.Precision.HIGHEST,
                      preferred_element_type=jnp.float32)
    loc1 = cs1 - 1.0
    loc2 = cs2 - 1.0 + jnp.sum(m1, axis=0, keepdims=True)
    k1m = m1 * (loc1 < C).astype(jnp.float32)
    k2m = m2 * (loc2 < C).astype(jnp.float32)
    p1 = jnp.sum(probs * k1m, axis=1, keepdims=True)
    p2 = jnp.sum(probs * k2m, axis=1, keepdims=True)
    denom = jnp.maximum(p1 + p2, jnp.finfo(jnp.float32).eps)
    w1 = p1 / denom * SCALE
    w2 = p2 / denom * SCALE
    kept1 = jnp.sum(k1m, axis=1, keepdims=True)
    kept2 = jnp.sum(k2m, axis=1, keepdims=True)
    ef = eidx.astype(jnp.float32)
    flat1 = jnp.where(kept1 > 0,
                      jnp.sum(m1 * (ef * C + loc1), axis=1, keepdims=True),
                      -1.0)
    flat2 = jnp.where(kept2 > 0,
                      jnp.sum(m2 * (ef * C + loc2), axis=1, keepdims=True),
                      -1.0)
    pad = jnp.zeros((S, 124), jnp.float32)
    route_ref[...] = jnp.concatenate([flat1, flat2, w1, w2, pad], axis=1)


def _k5(x2_ref, route_ref, buf_ref):
    e = pl.program_id(0)
    f1 = route_ref[:, 0:1].astype(jnp.int32)
    f2 = route_ref[:, 1:2].astype(jnp.int32)
    sl = jax.lax.broadcasted_iota(jnp.int32, (S, C), 1) + e * C
    p = jnp.logical_or(f1 == sl, f2 == sl).astype(jnp.float32)
    buf_ref[...] = jax.lax.dot_general(
        p, x2_ref[...], (((0,), (0,)), ((), ())),
        preferred_element_type=jnp.float32)


def _k6(buf_ref, w1_ref, b1_ref, w2_ref, b2_ref, out_ref):
    h = jnp.dot(buf_ref[...], w1_ref[0],
                preferred_element_type=jnp.float32) + b1_ref[0]
    h = h * 0.5 * (1.0 + jax.lax.erf(h * (1.0 / math.sqrt(2.0))))
    out_ref[...] = jnp.dot(h, w2_ref[0],
                           preferred_element_type=jnp.float32) + b2_ref[0]


def _k7(route_ref, h2f_ref, res_ref, out_ref):
    f1 = route_ref[:, 0:1].astype(jnp.int32)
    f2 = route_ref[:, 1:2].astype(jnp.int32)
    w1 = route_ref[:, 2:3]
    w2 = route_ref[:, 3:4]
    sl = jax.lax.broadcasted_iota(jnp.int32, (TT, E * C), 1)
    ce = (w1 * (f1 == sl).astype(jnp.float32)
          + w2 * (f2 == sl).astype(jnp.float32))
    out_ref[...] = res_ref[...] + jnp.dot(
        ce, h2f_ref[...], preferred_element_type=jnp.float32)


def kernel(hidden_states, ln1_g, ln1_b, q_w, q_b, k_w, k_b, v_w, v_b,
           o_w, o_b, ln2_g, ln2_b, router_w, w1, b1, w2, b2):
    f32 = jnp.float32
    h = hidden_states.reshape(S, D)
    r2 = lambda a: a.reshape(1, D)

    nt = S // TT
    q, k, v = pl.pallas_call(
        _k1,
        grid=(nt,),
        in_specs=[
            pl.BlockSpec((TT, D), lambda i: (i, 0)),
            pl.BlockSpec((1, D), lambda i: (0, 0)),
            pl.BlockSpec((1, D), lambda i: (0, 0)),
            pl.BlockSpec((D, D), lambda i: (0, 0)),
            pl.BlockSpec((1, D), lambda i: (0, 0)),
            pl.BlockSpec((D, D), lambda i: (0, 0)),
            pl.BlockSpec((1, D), lambda i: (0, 0)),
            pl.BlockSpec((D, D), lambda i: (0, 0)),
            pl.BlockSpec((1, D), lambda i: (0, 0)),
        ],
        out_specs=[pl.BlockSpec((H, TT, DH), lambda i: (0, i, 0))] * 3,
        out_shape=[jax.ShapeDtypeStruct((H, S, DH), f32)] * 3,
        interpret=_INTERPRET,
    )(h, r2(ln1_g), r2(ln1_b), q_w, r2(q_b), k_w, r2(k_b), v_w, r2(v_b))

    ao = pl.pallas_call(
        _k2,
        grid=(H, S // QT),
        in_specs=[
            pl.BlockSpec((1, QT, DH), lambda hh, qi: (hh, qi, 0)),
            pl.BlockSpec((1, S, DH), lambda hh, qi: (hh, 0, 0)),
            pl.BlockSpec((1, S, DH), lambda hh, qi: (hh, 0, 0)),
        ],
        out_specs=pl.BlockSpec((1, QT, DH), lambda hh, qi: (hh, qi, 0)),
        out_shape=jax.ShapeDtypeStruct((H, S, DH), f32),
        interpret=_INTERPRET,
    )(q, k, v)

    h2, x2, lg = pl.pallas_call(
        _k3,
        grid=(nt,),
        in_specs=[
            pl.BlockSpec((H, TT, DH), lambda i: (0, i, 0)),
            pl.BlockSpec((D, D), lambda i: (0, 0)),
            pl.BlockSpec((1, D), lambda i: (0, 0)),
            pl.BlockSpec((TT, D), lambda i: (i, 0)),
            pl.BlockSpec((1, D), lambda i: (0, 0)),
            pl.BlockSpec((1, D), lambda i: (0, 0)),
            pl.BlockSpec((E, D), lambda i: (0, 0)),
        ],
        out_specs=[
            pl.BlockSpec((TT, D), lambda i: (i, 0)),
            pl.BlockSpec((TT, D), lambda i: (i, 0)),
            pl.BlockSpec((TT, E), lambda i: (i, 0)),
        ],
        out_shape=[
            jax.ShapeDtypeStruct((S, D), f32),
            jax.ShapeDtypeStruct((S, D), f32),
            jax.ShapeDtypeStruct((S, E), f32),
        ],
        interpret=_INTERPRET,
    )(ao, o_w, r2(o_b), h, r2(ln2_g), r2(ln2_b), router_w)

    route = pl.pallas_call(
        _k4,
        in_specs=[pl.BlockSpec((S, E), lambda: (0, 0))],
        out_specs=pl.BlockSpec((S, 128), lambda: (0, 0)),
        out_shape=jax.ShapeDtypeStruct((S, 128), f32),
        interpret=_INTERPRET,
    )(lg)

    buf = pl.pallas_call(
        _k5,
        grid=(E,),
        in_specs=[
            pl.BlockSpec((S, D), lambda e: (0, 0)),
            pl.BlockSpec((S, 128), lambda e: (0, 0)),
        ],
        out_specs=pl.BlockSpec((C, D), lambda e: (e, 0)),
        out_shape=jax.ShapeDtypeStruct((E * C, D), f32),
        interpret=_INTERPRET,
    )(x2, route)

    h2f = pl.pallas_call(
        _k6,
        grid=(E,),
        in_specs=[
            pl.BlockSpec((C, D), lambda e: (e, 0)),
            pl.BlockSpec((1, D, DFF), lambda e: (e, 0, 0)),
            pl.BlockSpec((1, 1, DFF), lambda e: (e, 0, 0)),
            pl.BlockSpec((1, DFF, D), lambda e: (e, 0, 0)),
            pl.BlockSpec((1, 1, D), lambda e: (e, 0, 0)),
        ],
        out_specs=pl.BlockSpec((C, D), lambda e: (e, 0)),
        out_shape=jax.ShapeDtypeStruct((E * C, D), f32),
        interpret=_INTERPRET,
    )(buf, w1, b1.reshape(E, 1, DFF), w2, b2.reshape(E, 1, D))

    out = pl.pallas_call(
        _k7,
        grid=(nt,),
        in_specs=[
            pl.BlockSpec((TT, 128), lambda i: (i, 0)),
            pl.BlockSpec((E * C, D), lambda i: (0, 0)),
            pl.BlockSpec((TT, D), lambda i: (i, 0)),
        ],
        out_specs=pl.BlockSpec((TT, D), lambda i: (i, 0)),
        out_shape=jax.ShapeDtypeStruct((S, D), f32),
        interpret=_INTERPRET,
    )(route, h2f, h2)

    return out.reshape(1, S, D)


# speedup vs baseline: 1.3512x; 1.1545x over previous
import math

import jax
import jax.numpy as jnp
from jax.experimental import pallas as pl

S, D = 2048, 768
H, DH = 12, 64
E, DFF = 16, 3072
C = 2 * math.ceil(S / E)
LN_EPS = 1e-5
NEG = -1e30
SCALE = 0.8

_INTERPRET = False

QT = 1024
TT = 256


def _ln(x, g, b):
    mu = jnp.mean(x, axis=-1, keepdims=True)
    xc = x - mu
    var = jnp.mean(xc * xc, axis=-1, keepdims=True)
    return xc * jax.lax.rsqrt(var + LN_EPS) * g + b


def _dot_t(x, w):
    return jax.lax.dot_general(x, w, (((1,), (1,)), ((), ())),
                               preferred_element_type=jnp.float32)


def _k1(h_ref, g_ref, b_ref, qw_ref, qb_ref, kw_ref, kb_ref, vw_ref, vb_ref,
        q_ref, k_ref, v_ref):
    x = _ln(h_ref[...], g_ref[...], b_ref[...])
    q = (_dot_t(x, qw_ref[...]) + qb_ref[...]) * (1.0 / math.sqrt(DH))
    k = _dot_t(x, kw_ref[...]) + kb_ref[...]
    v = _dot_t(x, vw_ref[...]) + vb_ref[...]
    for h in range(H):
        q_ref[h] = q[:, DH * h:DH * (h + 1)]
        k_ref[h] = k[:, DH * h:DH * (h + 1)]
        v_ref[h] = v[:, DH * h:DH * (h + 1)]


def _k2(q_ref, k_ref, v_ref, o_ref, *, qoff):
    kw = k_ref.shape[1]
    s = _dot_t(q_ref[0], k_ref[0])
    ri = jax.lax.broadcasted_iota(jnp.int32, (QT, kw), 0) + qoff
    ci = jax.lax.broadcasted_iota(jnp.int32, (QT, kw), 1)
    s = jnp.where(ri >= ci, s, NEG)
    m = jnp.max(s, axis=1, keepdims=True)
    p = jnp.exp(s - m)
    r = 1.0 / jnp.sum(p, axis=1, keepdims=True)
    o_ref[0] = jnp.dot(p, v_ref[0], preferred_element_type=jnp.float32) * r


def _k3(ao_ref, ow_ref, ob_ref, res_ref, g_ref, b_ref, rw_ref,
        h2_ref, x2_ref, lg_ref):
    ao = jnp.concatenate([ao_ref[h] for h in range(H)], axis=1)
    h2 = _dot_t(ao, ow_ref[...]) + ob_ref[...] + res_ref[...]
    h2_ref[...] = h2
    x2 = _ln(h2, g_ref[...], b_ref[...])
    x2_ref[...] = x2
    lg_ref[...] = _dot_t(x2, rw_ref[...])


def _k4(lg_ref, route_ref):
    lg = lg_ref[...]
    eidx = jax.lax.broadcasted_iota(jnp.int32, (S, E), 1)
    mx1 = jnp.max(lg, axis=1, keepdims=True)
    t1 = jnp.min(jnp.where(lg == mx1, eidx, E), axis=1, keepdims=True)
    m1 = (eidx == t1).astype(jnp.float32)
    lg2 = jnp.where(m1 > 0, NEG, lg)
    mx2 = jnp.max(lg2, axis=1, keepdims=True)
    t2 = jnp.min(jnp.where(lg2 == mx2, eidx, E), axis=1, keepdims=True)
    m2 = (eidx == t2).astype(jnp.float32)
    ex = jnp.exp(lg - mx1)
    probs = ex / jnp.sum(ex, axis=1, keepdims=True)
    ri = jax.lax.broadcasted_iota(jnp.int32, (S, S), 0)
    ci = jax.lax.broadcasted_iota(jnp.int32, (S, S), 1)
    lt = (ri >= ci).astype(jnp.float32)
    cs1 = jax.lax.dot(lt, m1, precision=jax.lax.Precision.HIGHEST,
                      preferred_element_type=jnp.float32)
    cs2 = jax.lax.dot(lt, m2, precision=jax.lax.Precision.HIGHEST,
                      preferred_element_type=jnp.float32)
    loc1 = cs1 - 1.0
    loc2 = cs2 - 1.0 + jnp.sum(m1, axis=0, keepdims=True)
    k1m = m1 * (loc1 < C).astype(jnp.float32)
    k2m = m2 * (loc2 < C).astype(jnp.float32)
    p1 = jnp.sum(probs * k1m, axis=1, keepdims=True)
    p2 = jnp.sum(probs * k2m, axis=1, keepdims=True)
    denom = jnp.maximum(p1 + p2, jnp.finfo(jnp.float32).eps)
    w1 = p1 / denom * SCALE
    w2 = p2 / denom * SCALE
    kept1 = jnp.sum(k1m, axis=1, keepdims=True)
    kept2 = jnp.sum(k2m, axis=1, keepdims=True)
    ef = eidx.astype(jnp.float32)
    flat1 = jnp.where(kept1 > 0,
                      jnp.sum(m1 * (ef * C + loc1), axis=1, keepdims=True),
                      -1.0)
    flat2 = jnp.where(kept2 > 0,
                      jnp.sum(m2 * (ef * C + loc2), axis=1, keepdims=True),
                      -1.0)
    pad = jnp.zeros((S, 124), jnp.float32)
    route_ref[...] = jnp.concatenate([flat1, flat2, w1, w2, pad], axis=1)


def _k5(x2_ref, route_ref, buf_ref):
    e = pl.program_id(0)
    f1 = route_ref[:, 0:1].astype(jnp.int32)
    f2 = route_ref[:, 1:2].astype(jnp.int32)
    sl = jax.lax.broadcasted_iota(jnp.int32, (S, C), 1) + e * C
    p = jnp.logical_or(f1 == sl, f2 == sl).astype(jnp.float32)
    buf_ref[...] = jax.lax.dot_general(
        p, x2_ref[...], (((0,), (0,)), ((), ())),
        preferred_element_type=jnp.float32)


def _k6(buf_ref, w1_ref, b1_ref, w2_ref, b2_ref, out_ref):
    h = jnp.dot(buf_ref[...], w1_ref[0],
                preferred_element_type=jnp.float32) + b1_ref[0]
    h = h * 0.5 * (1.0 + jax.lax.erf(h * (1.0 / math.sqrt(2.0))))
    out_ref[...] = jnp.dot(h, w2_ref[0],
                           preferred_element_type=jnp.float32) + b2_ref[0]


def _k7(route_ref, h2f_ref, res_ref, out_ref):
    f1 = route_ref[:, 0:1].astype(jnp.int32)
    f2 = route_ref[:, 1:2].astype(jnp.int32)
    w1 = route_ref[:, 2:3]
    w2 = route_ref[:, 3:4]
    sl = jax.lax.broadcasted_iota(jnp.int32, (TT, E * C), 1)
    ce = (w1 * (f1 == sl).astype(jnp.float32)
          + w2 * (f2 == sl).astype(jnp.float32))
    out_ref[...] = res_ref[...] + jnp.dot(
        ce, h2f_ref[...], preferred_element_type=jnp.float32)


def kernel(hidden_states, ln1_g, ln1_b, q_w, q_b, k_w, k_b, v_w, v_b,
           o_w, o_b, ln2_g, ln2_b, router_w, w1, b1, w2, b2):
    f32 = jnp.float32
    h = hidden_states.reshape(S, D)
    r2 = lambda a: a.reshape(1, D)

    nt = S // TT
    q, k, v = pl.pallas_call(
        _k1,
        grid=(nt,),
        in_specs=[
            pl.BlockSpec((TT, D), lambda i: (i, 0)),
            pl.BlockSpec((1, D), lambda i: (0, 0)),
            pl.BlockSpec((1, D), lambda i: (0, 0)),
            pl.BlockSpec((D, D), lambda i: (0, 0)),
            pl.BlockSpec((1, D), lambda i: (0, 0)),
            pl.BlockSpec((D, D), lambda i: (0, 0)),
            pl.BlockSpec((1, D), lambda i: (0, 0)),
            pl.BlockSpec((D, D), lambda i: (0, 0)),
            pl.BlockSpec((1, D), lambda i: (0, 0)),
        ],
        out_specs=[pl.BlockSpec((H, TT, DH), lambda i: (0, i, 0))] * 3,
        out_shape=[jax.ShapeDtypeStruct((H, S, DH), f32)] * 3,
        interpret=_INTERPRET,
    )(h, r2(ln1_g), r2(ln1_b), q_w, r2(q_b), k_w, r2(k_b), v_w, r2(v_b))

    import functools as _ft
    ao_lo = pl.pallas_call(
        _ft.partial(_k2, qoff=0),
        grid=(H,),
        in_specs=[
            pl.BlockSpec((1, QT, DH), lambda hh: (hh, 0, 0)),
            pl.BlockSpec((1, QT, DH), lambda hh: (hh, 0, 0)),
            pl.BlockSpec((1, QT, DH), lambda hh: (hh, 0, 0)),
        ],
        out_specs=pl.BlockSpec((1, QT, DH), lambda hh: (hh, 0, 0)),
        out_shape=jax.ShapeDtypeStruct((H, QT, DH), f32),
        interpret=_INTERPRET,
    )(q, k, v)
    ao_hi = pl.pallas_call(
        _ft.partial(_k2, qoff=QT),
        grid=(H,),
        in_specs=[
            pl.BlockSpec((1, QT, DH), lambda hh: (hh, 1, 0)),
            pl.BlockSpec((1, S, DH), lambda hh: (hh, 0, 0)),
            pl.BlockSpec((1, S, DH), lambda hh: (hh, 0, 0)),
        ],
        out_specs=pl.BlockSpec((1, QT, DH), lambda hh: (hh, 0, 0)),
        out_shape=jax.ShapeDtypeStruct((H, QT, DH), f32),
        interpret=_INTERPRET,
    )(q, k, v)
    ao = jnp.concatenate([ao_lo, ao_hi], axis=1)

    h2, x2, lg = pl.pallas_call(
        _k3,
        grid=(nt,),
        in_specs=[
            pl.BlockSpec((H, TT, DH), lambda i: (0, i, 0)),
            pl.BlockSpec((D, D), lambda i: (0, 0)),
            pl.BlockSpec((1, D), lambda i: (0, 0)),
            pl.BlockSpec((TT, D), lambda i: (i, 0)),
            pl.BlockSpec((1, D), lambda i: (0, 0)),
            pl.BlockSpec((1, D), lambda i: (0, 0)),
            pl.BlockSpec((E, D), lambda i: (0, 0)),
        ],
        out_specs=[
            pl.BlockSpec((TT, D), lambda i: (i, 0)),
            pl.BlockSpec((TT, D), lambda i: (i, 0)),
            pl.BlockSpec((TT, E), lambda i: (i, 0)),
        ],
        out_shape=[
            jax.ShapeDtypeStruct((S, D), f32),
            jax.ShapeDtypeStruct((S, D), f32),
            jax.ShapeDtypeStruct((S, E), f32),
        ],
        interpret=_INTERPRET,
    )(ao, o_w, r2(o_b), h, r2(ln2_g), r2(ln2_b), router_w)

    route = pl.pallas_call(
        _k4,
        in_specs=[pl.BlockSpec((S, E), lambda: (0, 0))],
        out_specs=pl.BlockSpec((S, 128), lambda: (0, 0)),
        out_shape=jax.ShapeDtypeStruct((S, 128), f32),
        interpret=_INTERPRET,
    )(lg)

    buf = pl.pallas_call(
        _k5,
        grid=(E,),
        in_specs=[
            pl.BlockSpec((S, D), lambda e: (0, 0)),
            pl.BlockSpec((S, 128), lambda e: (0, 0)),
        ],
        out_specs=pl.BlockSpec((C, D), lambda e: (e, 0)),
        out_shape=jax.ShapeDtypeStruct((E * C, D), f32),
        interpret=_INTERPRET,
    )(x2, route)

    h2f = pl.pallas_call(
        _k6,
        grid=(E,),
        in_specs=[
            pl.BlockSpec((C, D), lambda e: (e, 0)),
            pl.BlockSpec((1, D, DFF), lambda e: (e, 0, 0)),
            pl.BlockSpec((1, 1, DFF), lambda e: (e, 0, 0)),
            pl.BlockSpec((1, DFF, D), lambda e: (e, 0, 0)),
            pl.BlockSpec((1, 1, D), lambda e: (e, 0, 0)),
        ],
        out_specs=pl.BlockSpec((C, D), lambda e: (e, 0)),
        out_shape=jax.ShapeDtypeStruct((E * C, D), f32),
        interpret=_INTERPRET,
    )(buf, w1, b1.reshape(E, 1, DFF), w2, b2.reshape(E, 1, D))

    out = pl.pallas_call(
        _k7,
        grid=(nt,),
        in_specs=[
            pl.BlockSpec((TT, 128), lambda i: (i, 0)),
            pl.BlockSpec((E * C, D), lambda i: (0, 0)),
            pl.BlockSpec((TT, D), lambda i: (i, 0)),
        ],
        out_specs=pl.BlockSpec((TT, D), lambda i: (i, 0)),
        out_shape=jax.ShapeDtypeStruct((S, D), f32),
        interpret=_INTERPRET,
    )(route, h2f, h2)

    return out.reshape(1, S, D)
